# baseline (speedup 1.0000x reference)
import functools

import jax
import jax.numpy as jnp
from jax import lax
from jax.experimental import pallas as pl
from jax.experimental.pallas import tpu as pltpu
from jax.experimental.pallas import tpu_sc as plsc

_N = 10000
_E = 160000
_FIN = 128
_DE = 16
_H = 8
_B = 64
_T = 12
_STEPS = 3
_LAYERS = 3

_PW = 80
_EBLK = 2000
_NBLK = 2000
_CH = 128
_NCHUNK = _E // _CH
_NW = 32

_f32 = jnp.float32


def _dot(a, b):
    return jnp.dot(a, b, preferred_element_type=_f32)


def _pre(edge_attr, w1, b1, x, w2, b2, root, bias):
    nnb = _N // _NBLK

    def body(ea_ref, w1_ref, b1_ref, x_ref, w2_ref, b2_ref, root_ref,
             bias_ref, h0_ref, h1_ref, h2_ref, p_ref, r_ref):
        y = jnp.maximum(_dot(ea_ref[:], w1_ref[:]) + b1_ref[:], 0.0)
        h0_ref[:] = y[:, 0:_H]
        h1_ref[:] = y[:, _H:2 * _H]
        h2_ref[:] = y[:, 2 * _H:3 * _H]

        @pl.when(pl.program_id(0) < nnb)
        def _():
            xb = x_ref[:]
            yn = _dot(xb, w2_ref[:])
            z = _dot(xb, b2_ref[:])
            p_ref[:] = jnp.concatenate([yn, z, jnp.zeros_like(z)], axis=1)
            r_ref[:] = _dot(xb, root_ref[:]) + bias_ref[:]

    def nmap(i, nnb=nnb):
        return (jnp.minimum(i, nnb - 1), 0)

    return pl.pallas_call(
        body,
        grid=(_E // _EBLK,),
        in_specs=[
            pl.BlockSpec((_EBLK, _DE), lambda i: (i, 0)),
            pl.BlockSpec((_DE, 3 * _H), lambda i: (0, 0)),
            pl.BlockSpec((1, 3 * _H), lambda i: (0, 0)),
            pl.BlockSpec((_NBLK, _FIN), nmap),
            pl.BlockSpec((_FIN, _H * _H), lambda i: (0, 0)),
            pl.BlockSpec((_FIN, _H), lambda i: (0, 0)),
            pl.BlockSpec((_FIN, _H), lambda i: (0, 0)),
            pl.BlockSpec((1, _H), lambda i: (0, 0)),
        ],
        out_specs=[
            pl.BlockSpec((_EBLK, _H), lambda i: (i, 0)),
            pl.BlockSpec((_EBLK, _H), lambda i: (i, 0)),
            pl.BlockSpec((_EBLK, _H), lambda i: (i, 0)),
            pl.BlockSpec((_NBLK, _PW), nmap),
            pl.BlockSpec((_NBLK, _H), nmap),
        ],
        out_shape=[
            jax.ShapeDtypeStruct((_E, _H), _f32),
            jax.ShapeDtypeStruct((_E, _H), _f32),
            jax.ShapeDtypeStruct((_E, _H), _f32),
            jax.ShapeDtypeStruct((_N, _PW), _f32),
            jax.ShapeDtypeStruct((_N, _H), _f32),
        ],
    )(edge_attr, w1, b1, x, w2, b2, root, bias)


def _combine_mid(parts, cnt, r, w2, b2, root, bias):
    def body(p0_ref, p1_ref, c0_ref, c1_ref, r_ref,
             w2_ref, b2_ref, root_ref, bias_ref, p_ref, rn_ref):
        s = p0_ref[:] + p1_ref[:]
        c = c0_ref[:] + c1_ref[:]
        xb = jnp.maximum(s / jnp.maximum(c, 1.0) + r_ref[:], 0.0)
        y = _dot(xb, w2_ref[:])
        z = _dot(xb, b2_ref[:])
        p_ref[:] = jnp.concatenate([y, z, jnp.zeros_like(z)], axis=1)
        rn_ref[:] = _dot(xb, root_ref[:]) + bias_ref[:]

    nb = _N // _NBLK
    return pl.pallas_call(
        body,
        grid=(nb,),
        in_specs=[
            pl.BlockSpec((_NBLK, _H), lambda i: (i, 0)),
            pl.BlockSpec((_NBLK, _H), lambda i, nb=nb: (i + nb, 0)),
            pl.BlockSpec((_NBLK, _H), lambda i: (i, 0)),
            pl.BlockSpec((_NBLK, _H), lambda i, nb=nb: (i + nb, 0)),
            pl.BlockSpec((_NBLK, _H), lambda i: (i, 0)),
            pl.BlockSpec((_H, _H * _H), lambda i: (0, 0)),
            pl.BlockSpec((_H, _H), lambda i: (0, 0)),
            pl.BlockSpec((_H, _H), lambda i: (0, 0)),
            pl.BlockSpec((1, _H), lambda i: (0, 0)),
        ],
        out_specs=[
            pl.BlockSpec((_NBLK, _PW), lambda i: (i, 0)),
            pl.BlockSpec((_NBLK, _H), lambda i: (i, 0)),
        ],
        out_shape=[
            jax.ShapeDtypeStruct((_N, _PW), _f32),
            jax.ShapeDtypeStruct((_N, _H), _f32),
        ],
    )(parts, parts, cnt, cnt, r, w2, b2, root, bias)


def _set2set_body(parts_ref, cnt_ref, r_ref, b_ref, wih_ref, whh_ref,
                  lb_ref, lw_ref, lbo_ref, out_ref):
    s = parts_ref[0:_N, :] + parts_ref[_N:2 * _N, :]
    c = cnt_ref[0:_N, :] + cnt_ref[_N:2 * _N, :]
    x = jnp.maximum(s / jnp.maximum(c, 1.0) + r_ref[:], 0.0)
    oh = (b_ref[:] == lax.broadcasted_iota(jnp.int32, (_N, _B), 1)).astype(_f32)
    q_star = jnp.zeros((_B, 2 * _H), _f32)
    hs = jnp.zeros((_B, _H), _f32)
    cs = jnp.zeros((_B, _H), _f32)
    for _ in range(_STEPS):
        gates = _dot(q_star, wih_ref[:]) + _dot(hs, whh_ref[:]) + lb_ref[:]
        i = jax.nn.sigmoid(gates[:, 0:_H])
        f = jax.nn.sigmoid(gates[:, _H:2 * _H])
        g = jnp.tanh(gates[:, 2 * _H:3 * _H])
        o = jax.nn.sigmoid(gates[:, 3 * _H:4 * _H])
        cs = f * cs + i * g
        hs = o * jnp.tanh(cs)
        q = hs
        xq = lax.dot_general(x, q, (((1,), (1,)), ((), ())),
                             preferred_element_type=_f32)
        e = jnp.sum(xq * oh, axis=1, keepdims=True)
        emax = jnp.max(jnp.where(oh > 0, e, -jnp.inf), axis=0, keepdims=True)
        emax = jnp.where(jnp.isfinite(emax), emax, 0.0)
        a = jnp.exp(e - jnp.sum(oh * emax, axis=1, keepdims=True))
        denom = jnp.sum(oh * a, axis=0, keepdims=True)
        a = a / (jnp.sum(oh * denom, axis=1, keepdims=True) + 1e-16)
        r = lax.dot_general(oh, a * x, (((0,), (0,)), ((), ())),
                            preferred_element_type=_f32)
        q_star = jnp.concatenate([q, r], axis=1)
    out_ref[:] = _dot(q_star, lw_ref[:]) + lbo_ref[:]


_set2set = pl.pallas_call(
    _set2set_body,
    out_shape=jax.ShapeDtypeStruct((_B, _T), _f32),
)


_EP = _E // _NW
_FC = _EP // _CH
_TAIL = _EP - _FC * _CH
_EPAD = _EP + 8
_ZB = 640

_sc_mesh = plsc.VectorSubcoreMesh(core_axis_name="c", subcore_axis_name="s")
_sc_params = pltpu.CompilerParams(use_tc_tiling_on_sc=False,
                                  needs_layout_passes=False)


def _make_layer(with_cnt):
    outs = [jax.ShapeDtypeStruct((2 * _N, _H), _f32)]
    scratch = [
        pltpu.VMEM((_EPAD,), jnp.int32),
        pltpu.VMEM((_EPAD,), jnp.int32),
        pltpu.VMEM((_EPAD, _H), _f32),
        pltpu.VMEM((_CH, _PW), _f32),
        pltpu.VMEM((_CH, _PW), _f32),
        pltpu.VMEM((_CH, _H), _f32),
        pltpu.VMEM((_CH, _H), _f32),
        pltpu.VMEM((_CH,), jnp.int32),
        pltpu.VMEM((_CH,), jnp.int32),
        pltpu.VMEM((_TAIL,), jnp.int32),
        pltpu.VMEM_SHARED((_N, _H), _f32),
        pltpu.SemaphoreType.DMA,
        pltpu.SemaphoreType.DMA,
        pltpu.SemaphoreType.DMA,
        pltpu.SemaphoreType.DMA,
    ]
    if with_cnt:
        outs.append(jax.ShapeDtypeStruct((2 * _N, _H), _f32))
        scratch += [
            pltpu.VMEM((_CH, _H), _f32),
            pltpu.VMEM_SHARED((_N, _H), _f32),
        ]

    @functools.partial(
        pl.kernel,
        out_type=outs if with_cnt else outs[0],
        mesh=_sc_mesh,
        compiler_params=_sc_params,
        scratch_types=scratch,
    )
    def layer_k(p_hbm, src_hbm, dst_hbm, h_hbm, zeros_hbm, *rest):
        if with_cnt:
            (ones_hbm, out_hbm, cnt_hbm, src_v, dst_v, h_v, rows_a, rows_b,
             msg_a, msg_b, idxd_a, idxd_b, idxd8_v, acc_s,
             gsem_a, gsem_b, ssem_a, ssem_b, ones_v, cnt_s) = rest
        else:
            (out_hbm, src_v, dst_v, h_v, rows_a, rows_b,
             msg_a, msg_b, idxd_a, idxd_b, idxd8_v, acc_s,
             gsem_a, gsem_b, ssem_a, ssem_b) = rest
        core = lax.axis_index("c")
        sid = lax.axis_index("s")
        wid = sid * 2 + core
        gb = wid * _EP

        pltpu.sync_copy(src_hbm.at[pl.ds(gb, _EP)], src_v.at[pl.ds(0, _EP)])
        pltpu.sync_copy(dst_hbm.at[pl.ds(gb, _EP)], dst_v.at[pl.ds(0, _EP)])
        pltpu.sync_copy(h_hbm.at[pl.ds(gb, _EP)], h_v.at[pl.ds(0, _EP)])
        if with_cnt:
            pltpu.sync_copy(ones_hbm, ones_v)

        @pl.when(sid < 15)
        def _():
            pltpu.sync_copy(zeros_hbm.at[pl.ds(sid * _ZB, _ZB)],
                            acc_s.at[pl.ds(sid * _ZB, _ZB)])
            if with_cnt:
                pltpu.sync_copy(zeros_hbm.at[pl.ds(sid * _ZB, _ZB)],
                                cnt_s.at[pl.ds(sid * _ZB, _ZB)])

        @pl.when(sid == 15)
        def _():
            pltpu.sync_copy(zeros_hbm.at[pl.ds(15 * _ZB, _N - 15 * _ZB)],
                            acc_s.at[pl.ds(15 * _ZB, _N - 15 * _ZB)])
            if with_cnt:
                pltpu.sync_copy(zeros_hbm.at[pl.ds(15 * _ZB, _N - 15 * _ZB)],
                                cnt_s.at[pl.ds(15 * _ZB, _N - 15 * _ZB)])

        iota16 = lax.iota(jnp.int32, 16)

        def splat(c):
            return jnp.full((16,), c, jnp.int32)

        def gview(eb, n, rows):
            return (p_hbm.at[src_v.at[pl.ds(eb, n)]],
                    rows.at[pl.ds(0, n)] if n != _CH else rows)

        def fetch(eb, rows, sem):
            s, d = gview(eb, _CH, rows)
            pltpu.async_copy(s, d, sem)

        def contract(eb, rows, ngroups, msg):
            for g in range(ngroups):
                r = g * 16 + iota16
                erow = eb + g * 16 + iota16
                accs = [plsc.load_gather(rows, [r, splat(8 * _H + o)])
                        for o in range(_H)]
                for k in range(_H):
                    hk = plsc.load_gather(h_v, [erow, splat(k)])
                    for o in range(_H):
                        accs[o] = accs[o] + hk * plsc.load_gather(
                            rows, [r, splat(8 * k + o)])
                for o in range(_H):
                    plsc.store_scatter(msg, [r, splat(o)], accs[o])

        def drain_scatter(msg, idxd, ssem):
            pltpu.make_async_copy(msg, acc_s.at[idxd], ssem).wait()
            if with_cnt:
                pltpu.make_async_copy(ones_v, cnt_s.at[idxd], ssem).wait()

        def compute_store(eb, rows, msg, idxd, gsem, ssem, drain):
            s, d = gview(eb, _CH, rows)
            pltpu.make_async_copy(s, d, gsem).wait()
            if drain is True:
                drain_scatter(msg, idxd, ssem)
            else:
                @pl.when(drain)
                def _():
                    drain_scatter(msg, idxd, ssem)
            for g in range(_CH // 16):
                idxd[pl.ds(g * 16, 16)] = dst_v[pl.ds(eb + g * 16, 16)]
            contract(eb, rows, _CH // 16, msg)
            pltpu.async_copy(msg, acc_s.at[idxd], ssem, add=True)
            if with_cnt:
                pltpu.async_copy(ones_v, cnt_s.at[idxd], ssem, add=True)

        fetch(0, rows_a, gsem_a)
        fetch(_CH, rows_b, gsem_b)
        plsc.subcore_barrier()

        def body(jj, carry):
            eb0 = 2 * jj * _CH
            compute_store(eb0, rows_a, msg_a, idxd_a, gsem_a, ssem_a, jj >= 1)
            fetch(eb0 + 2 * _CH, rows_a, gsem_a)
            compute_store(eb0 + _CH, rows_b, msg_b, idxd_b, gsem_b, ssem_b,
                          jj >= 1)

            @pl.when(jj < (_FC - 3) // 2)
            def _():
                fetch(eb0 + 3 * _CH, rows_b, gsem_b)

            return carry

        lax.fori_loop(0, (_FC - 1) // 2, body, 0)
        compute_store((_FC - 1) * _CH, rows_a, msg_a, idxd_a, gsem_a, ssem_a,
                      True)
        drain_scatter(msg_a, idxd_a, ssem_a)
        drain_scatter(msg_b, idxd_b, ssem_b)

        tb = _FC * _CH
        s, d = gview(tb, _TAIL, rows_a)
        pltpu.async_copy(s, d, gsem_a).wait()
        plsc.store_scatter(idxd8_v, [iota16], dst_v[pl.ds(tb, 16)],
                           mask=iota16 < _TAIL)
        contract(tb, rows_a, 1, msg_a)
        pltpu.sync_copy(msg_a.at[pl.ds(0, _TAIL)], acc_s.at[idxd8_v], add=True)
        if with_cnt:
            pltpu.sync_copy(ones_v.at[pl.ds(0, _TAIL)], cnt_s.at[idxd8_v], add=True)

        plsc.subcore_barrier()

        @pl.when(sid < 15)
        def _():
            pltpu.sync_copy(acc_s.at[pl.ds(sid * _ZB, _ZB)],
                            out_hbm.at[pl.ds(core * _N + sid * _ZB, _ZB)])
            if with_cnt:
                pltpu.sync_copy(cnt_s.at[pl.ds(sid * _ZB, _ZB)],
                                cnt_hbm.at[pl.ds(core * _N + sid * _ZB, _ZB)])

        @pl.when(sid == 15)
        def _():
            pltpu.sync_copy(acc_s.at[pl.ds(15 * _ZB, _N - 15 * _ZB)],
                            out_hbm.at[pl.ds(core * _N + 15 * _ZB, _N - 15 * _ZB)])
            if with_cnt:
                pltpu.sync_copy(cnt_s.at[pl.ds(15 * _ZB, _N - 15 * _ZB)],
                                cnt_hbm.at[pl.ds(core * _N + 15 * _ZB, _N - 15 * _ZB)])

    return layer_k


_sc_layer_cnt = _make_layer(True)
_sc_layer = _make_layer(False)


def _w2r(p, l, fin):
    return p['en2_W%d' % l].reshape(_H, fin, _H).transpose(1, 0, 2).reshape(fin, _H * _H)


def kernel(x, edge_index, edge_attr, batch, params):
    p = params
    src = edge_index[0]
    dst = edge_index[1]

    w1cat = jnp.concatenate([p['en1_W%d' % l] for l in range(_LAYERS)], axis=1)
    b1cat = jnp.concatenate([p['en1_b%d' % l] for l in range(_LAYERS)]).reshape(1, 3 * _H)

    fins = [_FIN, _H, _H]
    h0, h1, h2, P, R = _pre(edge_attr, w1cat, b1cat,
                            x, _w2r(p, 0, _FIN), p['en2_b0'].reshape(_FIN, _H),
                            p['root0'], p['bias0'].reshape(1, _H))
    hs = [h0, h1, h2]

    zerosN = jnp.zeros((_N, _H), _f32)
    onesC = jnp.ones((_CH, _H), _f32)

    cnt = None
    for l in range(_LAYERS):
        if l == 0:
            parts, cnt = _sc_layer_cnt(P, src, dst, hs[l], zerosN, onesC)
        else:
            parts = _sc_layer(P, src, dst, hs[l], zerosN)
        if l < _LAYERS - 1:
            fin = fins[l + 1]
            P, R = _combine_mid(parts, cnt, R,
                                _w2r(p, l + 1, fin),
                                p['en2_b%d' % (l + 1)].reshape(fin, _H),
                                p['root%d' % (l + 1)],
                                p['bias%d' % (l + 1)].reshape(1, _H))

    return _set2set(parts, cnt, R, batch.reshape(_N, 1),
                    p['Wih'], p['Whh'], p['lstm_b'].reshape(1, 4 * _H),
                    p['lin_W'], p['lin_b'].reshape(1, _T))


# speedup vs baseline: 1.0127x; 1.0127x over previous
import functools

import jax
import jax.numpy as jnp
from jax import lax
from jax.experimental import pallas as pl
from jax.experimental.pallas import tpu as pltpu
from jax.experimental.pallas import tpu_sc as plsc

_N = 10000
_E = 160000
_FIN = 128
_DE = 16
_H = 8
_B = 64
_T = 12
_STEPS = 3
_LAYERS = 3

_PW = 80
_EBLK = 2000
_NBLK = 2000
_CH = 128
_NCHUNK = _E // _CH
_NW = 32

_f32 = jnp.float32


def _dot(a, b):
    return jnp.dot(a, b, preferred_element_type=_f32)


def _pre(edge_attr, w1, b1, x, w2, b2, root, bias):
    nnb = _N // _NBLK

    def body(ea_ref, w1_ref, b1_ref, x_ref, w2_ref, b2_ref, root_ref,
             bias_ref, h0_ref, h1_ref, h2_ref, p_ref, r_ref):
        y = jnp.maximum(_dot(ea_ref[:], w1_ref[:]) + b1_ref[:], 0.0)
        h0_ref[:] = y[:, 0:_H]
        h1_ref[:] = y[:, _H:2 * _H]
        h2_ref[:] = y[:, 2 * _H:3 * _H]

        @pl.when(pl.program_id(0) < nnb)
        def _():
            xb = x_ref[:]
            yn = _dot(xb, w2_ref[:])
            z = _dot(xb, b2_ref[:])
            p_ref[:] = jnp.concatenate([yn, z, jnp.zeros_like(z)], axis=1)
            r_ref[:] = _dot(xb, root_ref[:]) + bias_ref[:]

    def nmap(i, nnb=nnb):
        return (jnp.minimum(i, nnb - 1), 0)

    return pl.pallas_call(
        body,
        grid=(_E // _EBLK,),
        in_specs=[
            pl.BlockSpec((_EBLK, _DE), lambda i: (i, 0)),
            pl.BlockSpec((_DE, 3 * _H), lambda i: (0, 0)),
            pl.BlockSpec((1, 3 * _H), lambda i: (0, 0)),
            pl.BlockSpec((_NBLK, _FIN), nmap),
            pl.BlockSpec((_FIN, _H * _H), lambda i: (0, 0)),
            pl.BlockSpec((_FIN, _H), lambda i: (0, 0)),
            pl.BlockSpec((_FIN, _H), lambda i: (0, 0)),
            pl.BlockSpec((1, _H), lambda i: (0, 0)),
        ],
        out_specs=[
            pl.BlockSpec((_EBLK, _H), lambda i: (i, 0)),
            pl.BlockSpec((_EBLK, _H), lambda i: (i, 0)),
            pl.BlockSpec((_EBLK, _H), lambda i: (i, 0)),
            pl.BlockSpec((_NBLK, _PW), nmap),
            pl.BlockSpec((_NBLK, _H), nmap),
        ],
        out_shape=[
            jax.ShapeDtypeStruct((_E, _H), _f32),
            jax.ShapeDtypeStruct((_E, _H), _f32),
            jax.ShapeDtypeStruct((_E, _H), _f32),
            jax.ShapeDtypeStruct((_N, _PW), _f32),
            jax.ShapeDtypeStruct((_N, _H), _f32),
        ],
    )(edge_attr, w1, b1, x, w2, b2, root, bias)


def _combine_mid(parts, cnt, r, w2, b2, root, bias):
    def body(p0_ref, p1_ref, c0_ref, c1_ref, r_ref,
             w2_ref, b2_ref, root_ref, bias_ref, p_ref, rn_ref):
        s = p0_ref[:] + p1_ref[:]
        c = c0_ref[:] + c1_ref[:]
        xb = jnp.maximum(s / jnp.maximum(c, 1.0) + r_ref[:], 0.0)
        y = _dot(xb, w2_ref[:])
        z = _dot(xb, b2_ref[:])
        p_ref[:] = jnp.concatenate([y, z, jnp.zeros_like(z)], axis=1)
        rn_ref[:] = _dot(xb, root_ref[:]) + bias_ref[:]

    nb = _N // _NBLK
    return pl.pallas_call(
        body,
        grid=(nb,),
        in_specs=[
            pl.BlockSpec((_NBLK, _H), lambda i: (i, 0)),
            pl.BlockSpec((_NBLK, _H), lambda i, nb=nb: (i + nb, 0)),
            pl.BlockSpec((_NBLK, _H), lambda i: (i, 0)),
            pl.BlockSpec((_NBLK, _H), lambda i, nb=nb: (i + nb, 0)),
            pl.BlockSpec((_NBLK, _H), lambda i: (i, 0)),
            pl.BlockSpec((_H, _H * _H), lambda i: (0, 0)),
            pl.BlockSpec((_H, _H), lambda i: (0, 0)),
            pl.BlockSpec((_H, _H), lambda i: (0, 0)),
            pl.BlockSpec((1, _H), lambda i: (0, 0)),
        ],
        out_specs=[
            pl.BlockSpec((_NBLK, _PW), lambda i: (i, 0)),
            pl.BlockSpec((_NBLK, _H), lambda i: (i, 0)),
        ],
        out_shape=[
            jax.ShapeDtypeStruct((_N, _PW), _f32),
            jax.ShapeDtypeStruct((_N, _H), _f32),
        ],
    )(parts, parts, cnt, cnt, r, w2, b2, root, bias)


def _set2set_body(parts_ref, cnt_ref, r_ref, b_ref, wih_ref, whh_ref,
                  lb_ref, lw_ref, lbo_ref, out_ref):
    s = parts_ref[0:_N, :] + parts_ref[_N:2 * _N, :]
    c = cnt_ref[0:_N, :] + cnt_ref[_N:2 * _N, :]
    x = jnp.maximum(s / jnp.maximum(c, 1.0) + r_ref[:], 0.0)
    oh = (b_ref[:] == lax.broadcasted_iota(jnp.int32, (_N, _B), 1)).astype(_f32)
    q_star = jnp.zeros((_B, 2 * _H), _f32)
    hs = jnp.zeros((_B, _H), _f32)
    cs = jnp.zeros((_B, _H), _f32)
    for _ in range(_STEPS):
        gates = _dot(q_star, wih_ref[:]) + _dot(hs, whh_ref[:]) + lb_ref[:]
        i = jax.nn.sigmoid(gates[:, 0:_H])
        f = jax.nn.sigmoid(gates[:, _H:2 * _H])
        g = jnp.tanh(gates[:, 2 * _H:3 * _H])
        o = jax.nn.sigmoid(gates[:, 3 * _H:4 * _H])
        cs = f * cs + i * g
        hs = o * jnp.tanh(cs)
        q = hs
        xq = lax.dot_general(x, q, (((1,), (1,)), ((), ())),
                             preferred_element_type=_f32)
        m = jnp.where(oh > 0, xq, -jnp.inf)
        emax = jnp.max(m, axis=0, keepdims=True)
        emax = jnp.where(jnp.isfinite(emax), emax, 0.0)
        a = jnp.exp(m - emax)
        denom = jnp.sum(a, axis=0, keepdims=True)
        a = a / (denom + 1e-16)
        r = lax.dot_general(a, x, (((0,), (0,)), ((), ())),
                            preferred_element_type=_f32)
        q_star = jnp.concatenate([q, r], axis=1)
    out_ref[:] = _dot(q_star, lw_ref[:]) + lbo_ref[:]


_set2set = pl.pallas_call(
    _set2set_body,
    out_shape=jax.ShapeDtypeStruct((_B, _T), _f32),
)


_EP = _E // _NW
_FC = _EP // _CH
_TAIL = _EP - _FC * _CH
_EPAD = _EP + 8
_ZB = 640

_sc_mesh = plsc.VectorSubcoreMesh(core_axis_name="c", subcore_axis_name="s")
_sc_params = pltpu.CompilerParams(use_tc_tiling_on_sc=False,
                                  needs_layout_passes=False)


def _make_layer(with_cnt):
    outs = [jax.ShapeDtypeStruct((2 * _N, _H), _f32)]
    scratch = [
        pltpu.VMEM((_EPAD,), jnp.int32),
        pltpu.VMEM((_EPAD,), jnp.int32),
        pltpu.VMEM((_EPAD, _H), _f32),
        pltpu.VMEM((_CH, _PW), _f32),
        pltpu.VMEM((_CH, _PW), _f32),
        pltpu.VMEM((_CH, _H), _f32),
        pltpu.VMEM((_CH, _H), _f32),
        pltpu.VMEM((_CH,), jnp.int32),
        pltpu.VMEM((_CH,), jnp.int32),
        pltpu.VMEM((_TAIL,), jnp.int32),
        pltpu.VMEM_SHARED((_N, _H), _f32),
        pltpu.SemaphoreType.DMA,
        pltpu.SemaphoreType.DMA,
        pltpu.SemaphoreType.DMA,
        pltpu.SemaphoreType.DMA,
    ]
    if with_cnt:
        outs.append(jax.ShapeDtypeStruct((2 * _N, _H), _f32))
        scratch += [
            pltpu.VMEM((_CH, _H), _f32),
            pltpu.VMEM_SHARED((_N, _H), _f32),
        ]

    @functools.partial(
        pl.kernel,
        out_type=outs if with_cnt else outs[0],
        mesh=_sc_mesh,
        compiler_params=_sc_params,
        scratch_types=scratch,
    )
    def layer_k(p_hbm, src_hbm, dst_hbm, h_hbm, zeros_hbm, *rest):
        if with_cnt:
            (ones_hbm, out_hbm, cnt_hbm, src_v, dst_v, h_v, rows_a, rows_b,
             msg_a, msg_b, idxd_a, idxd_b, idxd8_v, acc_s,
             gsem_a, gsem_b, ssem_a, ssem_b, ones_v, cnt_s) = rest
        else:
            (out_hbm, src_v, dst_v, h_v, rows_a, rows_b,
             msg_a, msg_b, idxd_a, idxd_b, idxd8_v, acc_s,
             gsem_a, gsem_b, ssem_a, ssem_b) = rest
        core = lax.axis_index("c")
        sid = lax.axis_index("s")
        wid = sid * 2 + core
        gb = wid * _EP

        pltpu.sync_copy(src_hbm.at[pl.ds(gb, _EP)], src_v.at[pl.ds(0, _EP)])
        pltpu.sync_copy(dst_hbm.at[pl.ds(gb, _EP)], dst_v.at[pl.ds(0, _EP)])
        pltpu.sync_copy(h_hbm.at[pl.ds(gb, _EP)], h_v.at[pl.ds(0, _EP)])
        if with_cnt:
            pltpu.sync_copy(ones_hbm, ones_v)

        @pl.when(sid < 15)
        def _():
            pltpu.sync_copy(zeros_hbm.at[pl.ds(sid * _ZB, _ZB)],
                            acc_s.at[pl.ds(sid * _ZB, _ZB)])
            if with_cnt:
                pltpu.sync_copy(zeros_hbm.at[pl.ds(sid * _ZB, _ZB)],
                                cnt_s.at[pl.ds(sid * _ZB, _ZB)])

        @pl.when(sid == 15)
        def _():
            pltpu.sync_copy(zeros_hbm.at[pl.ds(15 * _ZB, _N - 15 * _ZB)],
                            acc_s.at[pl.ds(15 * _ZB, _N - 15 * _ZB)])
            if with_cnt:
                pltpu.sync_copy(zeros_hbm.at[pl.ds(15 * _ZB, _N - 15 * _ZB)],
                                cnt_s.at[pl.ds(15 * _ZB, _N - 15 * _ZB)])

        iota16 = lax.iota(jnp.int32, 16)

        def splat(c):
            return jnp.full((16,), c, jnp.int32)

        def gview(eb, n, rows):
            return (p_hbm.at[src_v.at[pl.ds(eb, n)]],
                    rows.at[pl.ds(0, n)] if n != _CH else rows)

        def fetch(eb, rows, sem):
            s, d = gview(eb, _CH, rows)
            pltpu.async_copy(s, d, sem)

        def contract(eb, rows, ngroups, msg):
            for g in range(ngroups):
                r = g * 16 + iota16
                erow = eb + g * 16 + iota16
                accs = [plsc.load_gather(rows, [r, splat(8 * _H + o)])
                        for o in range(_H)]
                for k in range(_H):
                    hk = plsc.load_gather(h_v, [erow, splat(k)])
                    for o in range(_H):
                        accs[o] = accs[o] + hk * plsc.load_gather(
                            rows, [r, splat(8 * k + o)])
                for o in range(_H):
                    plsc.store_scatter(msg, [r, splat(o)], accs[o])

        def drain_scatter(msg, idxd, ssem):
            pltpu.make_async_copy(msg, acc_s.at[idxd], ssem).wait()
            if with_cnt:
                pltpu.make_async_copy(ones_v, cnt_s.at[idxd], ssem).wait()

        def compute_store(eb, rows, msg, idxd, gsem, ssem, drain):
            s, d = gview(eb, _CH, rows)
            pltpu.make_async_copy(s, d, gsem).wait()
            if drain is True:
                drain_scatter(msg, idxd, ssem)
            else:
                @pl.when(drain)
                def _():
                    drain_scatter(msg, idxd, ssem)
            for g in range(_CH // 16):
                idxd[pl.ds(g * 16, 16)] = dst_v[pl.ds(eb + g * 16, 16)]
            contract(eb, rows, _CH // 16, msg)
            pltpu.async_copy(msg, acc_s.at[idxd], ssem, add=True)
            if with_cnt:
                pltpu.async_copy(ones_v, cnt_s.at[idxd], ssem, add=True)

        fetch(0, rows_a, gsem_a)
        fetch(_CH, rows_b, gsem_b)
        plsc.subcore_barrier()

        def body(jj, carry):
            eb0 = 2 * jj * _CH
            compute_store(eb0, rows_a, msg_a, idxd_a, gsem_a, ssem_a, jj >= 1)
            fetch(eb0 + 2 * _CH, rows_a, gsem_a)
            compute_store(eb0 + _CH, rows_b, msg_b, idxd_b, gsem_b, ssem_b,
                          jj >= 1)

            @pl.when(jj < (_FC - 3) // 2)
            def _():
                fetch(eb0 + 3 * _CH, rows_b, gsem_b)

            return carry

        lax.fori_loop(0, (_FC - 1) // 2, body, 0)
        compute_store((_FC - 1) * _CH, rows_a, msg_a, idxd_a, gsem_a, ssem_a,
                      True)
        drain_scatter(msg_a, idxd_a, ssem_a)
        drain_scatter(msg_b, idxd_b, ssem_b)

        tb = _FC * _CH
        s, d = gview(tb, _TAIL, rows_a)
        pltpu.async_copy(s, d, gsem_a).wait()
        plsc.store_scatter(idxd8_v, [iota16], dst_v[pl.ds(tb, 16)],
                           mask=iota16 < _TAIL)
        contract(tb, rows_a, 1, msg_a)
        pltpu.sync_copy(msg_a.at[pl.ds(0, _TAIL)], acc_s.at[idxd8_v], add=True)
        if with_cnt:
            pltpu.sync_copy(ones_v.at[pl.ds(0, _TAIL)], cnt_s.at[idxd8_v], add=True)

        plsc.subcore_barrier()

        @pl.when(sid < 15)
        def _():
            pltpu.sync_copy(acc_s.at[pl.ds(sid * _ZB, _ZB)],
                            out_hbm.at[pl.ds(core * _N + sid * _ZB, _ZB)])
            if with_cnt:
                pltpu.sync_copy(cnt_s.at[pl.ds(sid * _ZB, _ZB)],
                                cnt_hbm.at[pl.ds(core * _N + sid * _ZB, _ZB)])

        @pl.when(sid == 15)
        def _():
            pltpu.sync_copy(acc_s.at[pl.ds(15 * _ZB, _N - 15 * _ZB)],
                            out_hbm.at[pl.ds(core * _N + 15 * _ZB, _N - 15 * _ZB)])
            if with_cnt:
                pltpu.sync_copy(cnt_s.at[pl.ds(15 * _ZB, _N - 15 * _ZB)],
                                cnt_hbm.at[pl.ds(core * _N + 15 * _ZB, _N - 15 * _ZB)])

    return layer_k


_sc_layer_cnt = _make_layer(True)
_sc_layer = _make_layer(False)


def _w2r(p, l, fin):
    return p['en2_W%d' % l].reshape(_H, fin, _H).transpose(1, 0, 2).reshape(fin, _H * _H)


def kernel(x, edge_index, edge_attr, batch, params):
    p = params
    src = edge_index[0]
    dst = edge_index[1]

    w1cat = jnp.concatenate([p['en1_W%d' % l] for l in range(_LAYERS)], axis=1)
    b1cat = jnp.concatenate([p['en1_b%d' % l] for l in range(_LAYERS)]).reshape(1, 3 * _H)

    fins = [_FIN, _H, _H]
    h0, h1, h2, P, R = _pre(edge_attr, w1cat, b1cat,
                            x, _w2r(p, 0, _FIN), p['en2_b0'].reshape(_FIN, _H),
                            p['root0'], p['bias0'].reshape(1, _H))
    hs = [h0, h1, h2]

    zerosN = jnp.zeros((_N, _H), _f32)
    onesC = jnp.ones((_CH, _H), _f32)

    cnt = None
    for l in range(_LAYERS):
        if l == 0:
            parts, cnt = _sc_layer_cnt(P, src, dst, hs[l], zerosN, onesC)
        else:
            parts = _sc_layer(P, src, dst, hs[l], zerosN)
        if l < _LAYERS - 1:
            fin = fins[l + 1]
            P, R = _combine_mid(parts, cnt, R,
                                _w2r(p, l + 1, fin),
                                p['en2_b%d' % (l + 1)].reshape(fin, _H),
                                p['root%d' % (l + 1)],
                                p['bias%d' % (l + 1)].reshape(1, _H))

    return _set2set(parts, cnt, R, batch.reshape(_N, 1),
                    p['Wih'], p['Whh'], p['lstm_b'].reshape(1, 4 * _H),
                    p['lin_W'], p['lin_b'].reshape(1, _T))


# speedup vs baseline: 1.0332x; 1.0203x over previous
import functools

import jax
import jax.numpy as jnp
from jax import lax
from jax.experimental import pallas as pl
from jax.experimental.pallas import tpu as pltpu
from jax.experimental.pallas import tpu_sc as plsc

_N = 10000
_E = 160000
_FIN = 128
_DE = 16
_H = 8
_B = 64
_T = 12
_STEPS = 3
_LAYERS = 3

_PW = 80
_EBLK = 2000
_NBLK = 2000
_CH = 128
_NCHUNK = _E // _CH
_NW = 32

_f32 = jnp.float32


def _dot(a, b):
    return jnp.dot(a, b, preferred_element_type=_f32)


def _pre(edge_attr, w1, b1, x, w2, b2, root, bias):
    nnb = _N // _NBLK

    def body(ea_ref, w1_ref, b1_ref, x_ref, w2_ref, b2_ref, root_ref,
             bias_ref, h0_ref, h1_ref, h2_ref, p_ref, r_ref):
        y = jnp.maximum(_dot(ea_ref[:], w1_ref[:]) + b1_ref[:], 0.0)
        h0_ref[:] = y[:, 0:_H]
        h1_ref[:] = y[:, _H:2 * _H]
        h2_ref[:] = y[:, 2 * _H:3 * _H]

        @pl.when(pl.program_id(0) < nnb)
        def _():
            xb = x_ref[:]
            yn = _dot(xb, w2_ref[:])
            z = _dot(xb, b2_ref[:])
            p_ref[:] = jnp.concatenate([yn, z, jnp.zeros_like(z)], axis=1)
            r_ref[:] = _dot(xb, root_ref[:]) + bias_ref[:]

    def nmap(i, nnb=nnb):
        return (jnp.minimum(i, nnb - 1), 0)

    return pl.pallas_call(
        body,
        grid=(_E // _EBLK,),
        in_specs=[
            pl.BlockSpec((_EBLK, _DE), lambda i: (i, 0)),
            pl.BlockSpec((_DE, 3 * _H), lambda i: (0, 0)),
            pl.BlockSpec((1, 3 * _H), lambda i: (0, 0)),
            pl.BlockSpec((_NBLK, _FIN), nmap),
            pl.BlockSpec((_FIN, _H * _H), lambda i: (0, 0)),
            pl.BlockSpec((_FIN, _H), lambda i: (0, 0)),
            pl.BlockSpec((_FIN, _H), lambda i: (0, 0)),
            pl.BlockSpec((1, _H), lambda i: (0, 0)),
        ],
        out_specs=[
            pl.BlockSpec((_EBLK, _H), lambda i: (i, 0)),
            pl.BlockSpec((_EBLK, _H), lambda i: (i, 0)),
            pl.BlockSpec((_EBLK, _H), lambda i: (i, 0)),
            pl.BlockSpec((_NBLK, _PW), nmap),
            pl.BlockSpec((_NBLK, _H), nmap),
        ],
        out_shape=[
            jax.ShapeDtypeStruct((_E, _H), _f32),
            jax.ShapeDtypeStruct((_E, _H), _f32),
            jax.ShapeDtypeStruct((_E, _H), _f32),
            jax.ShapeDtypeStruct((_N, _PW), _f32),
            jax.ShapeDtypeStruct((_N, _H), _f32),
        ],
    )(edge_attr, w1, b1, x, w2, b2, root, bias)


def _combine_mid(parts, cnt, r, w2, b2, root, bias):
    def body(p0_ref, p1_ref, c0_ref, c1_ref, r_ref,
             w2_ref, b2_ref, root_ref, bias_ref, p_ref, rn_ref):
        s = p0_ref[:] + p1_ref[:]
        c = c0_ref[:] + c1_ref[:]
        xb = jnp.maximum(s / jnp.maximum(c, 1.0) + r_ref[:], 0.0)
        y = _dot(xb, w2_ref[:])
        z = _dot(xb, b2_ref[:])
        p_ref[:] = jnp.concatenate([y, z, jnp.zeros_like(z)], axis=1)
        rn_ref[:] = _dot(xb, root_ref[:]) + bias_ref[:]

    nb = _N // _NBLK
    return pl.pallas_call(
        body,
        grid=(nb,),
        in_specs=[
            pl.BlockSpec((_NBLK, _H), lambda i: (i, 0)),
            pl.BlockSpec((_NBLK, _H), lambda i, nb=nb: (i + nb, 0)),
            pl.BlockSpec((_NBLK, _H), lambda i: (i, 0)),
            pl.BlockSpec((_NBLK, _H), lambda i, nb=nb: (i + nb, 0)),
            pl.BlockSpec((_NBLK, _H), lambda i: (i, 0)),
            pl.BlockSpec((_H, _H * _H), lambda i: (0, 0)),
            pl.BlockSpec((_H, _H), lambda i: (0, 0)),
            pl.BlockSpec((_H, _H), lambda i: (0, 0)),
            pl.BlockSpec((1, _H), lambda i: (0, 0)),
        ],
        out_specs=[
            pl.BlockSpec((_NBLK, _PW), lambda i: (i, 0)),
            pl.BlockSpec((_NBLK, _H), lambda i: (i, 0)),
        ],
        out_shape=[
            jax.ShapeDtypeStruct((_N, _PW), _f32),
            jax.ShapeDtypeStruct((_N, _H), _f32),
        ],
    )(parts, parts, cnt, cnt, r, w2, b2, root, bias)


def _set2set_body(parts_ref, cnt_ref, r_ref, b_ref, wih_ref, whh_ref,
                  lb_ref, lw_ref, lbo_ref, out_ref):
    s = parts_ref[0:_N, :] + parts_ref[_N:2 * _N, :]
    c = cnt_ref[0:_N, :] + cnt_ref[_N:2 * _N, :]
    x = jnp.maximum(s / jnp.maximum(c, 1.0) + r_ref[:], 0.0)
    oh = (b_ref[:] == lax.broadcasted_iota(jnp.int32, (_N, _B), 1)).astype(_f32)
    q_star = jnp.zeros((_B, 2 * _H), _f32)
    hs = jnp.zeros((_B, _H), _f32)
    cs = jnp.zeros((_B, _H), _f32)
    for _ in range(_STEPS):
        gates = _dot(q_star, wih_ref[:]) + _dot(hs, whh_ref[:]) + lb_ref[:]
        i = jax.nn.sigmoid(gates[:, 0:_H])
        f = jax.nn.sigmoid(gates[:, _H:2 * _H])
        g = jnp.tanh(gates[:, 2 * _H:3 * _H])
        o = jax.nn.sigmoid(gates[:, 3 * _H:4 * _H])
        cs = f * cs + i * g
        hs = o * jnp.tanh(cs)
        q = hs
        xq = lax.dot_general(x, q, (((1,), (1,)), ((), ())),
                             preferred_element_type=_f32)
        m = jnp.where(oh > 0, xq, -jnp.inf)
        emax = jnp.max(m, axis=0, keepdims=True)
        emax = jnp.where(jnp.isfinite(emax), emax, 0.0)
        a = jnp.exp(m - emax)
        denom = jnp.sum(a, axis=0, keepdims=True)
        a = a / (denom + 1e-16)
        r = lax.dot_general(a, x, (((0,), (0,)), ((), ())),
                            preferred_element_type=_f32)
        q_star = jnp.concatenate([q, r], axis=1)
    out_ref[:] = _dot(q_star, lw_ref[:]) + lbo_ref[:]


_set2set = pl.pallas_call(
    _set2set_body,
    out_shape=jax.ShapeDtypeStruct((_B, _T), _f32),
)


_EP = _E // _NW
_FC = _EP // _CH
_TAIL = _EP - _FC * _CH
_EPAD = _EP + 8
_ZB = 640

_sc_mesh = plsc.VectorSubcoreMesh(core_axis_name="c", subcore_axis_name="s")
_sc_params = pltpu.CompilerParams(use_tc_tiling_on_sc=False,
                                  needs_layout_passes=False)


def _make_layer(with_cnt):
    outs = [jax.ShapeDtypeStruct((2 * _N, _H), _f32)]
    scratch = [
        pltpu.VMEM((_EPAD,), jnp.int32),
        pltpu.VMEM((_EPAD,), jnp.int32),
        pltpu.VMEM((_EPAD, _H), _f32),
        pltpu.VMEM((_CH, _PW), _f32),
        pltpu.VMEM((_CH, _PW), _f32),
        pltpu.VMEM((_CH, _H), _f32),
        pltpu.VMEM((_CH, _H), _f32),
        pltpu.VMEM((_CH,), jnp.int32),
        pltpu.VMEM((_CH,), jnp.int32),
        pltpu.VMEM((_TAIL,), jnp.int32),
        pltpu.VMEM_SHARED((_N, _H), _f32),
        pltpu.SemaphoreType.DMA,
        pltpu.SemaphoreType.DMA,
        pltpu.SemaphoreType.DMA,
        pltpu.SemaphoreType.DMA,
        pltpu.SemaphoreType.DMA,
    ]
    if with_cnt:
        outs.append(jax.ShapeDtypeStruct((2 * _N, _H), _f32))
        scratch += [
            pltpu.VMEM((_CH, _H), _f32),
            pltpu.VMEM_SHARED((_N, _H), _f32),
        ]

    @functools.partial(
        pl.kernel,
        out_type=outs if with_cnt else outs[0],
        mesh=_sc_mesh,
        compiler_params=_sc_params,
        scratch_types=scratch,
    )
    def layer_k(p_hbm, src_hbm, dst_hbm, h_hbm, zeros_hbm, *rest):
        if with_cnt:
            (ones_hbm, out_hbm, cnt_hbm, src_v, dst_v, h_v, rows_a, rows_b,
             msg_a, msg_b, idxd_a, idxd_b, idxd8_v, acc_s,
             gsem_a, gsem_b, ssem_a, ssem_b, stsem, ones_v, cnt_s) = rest
        else:
            (out_hbm, src_v, dst_v, h_v, rows_a, rows_b,
             msg_a, msg_b, idxd_a, idxd_b, idxd8_v, acc_s,
             gsem_a, gsem_b, ssem_a, ssem_b, stsem) = rest
        core = lax.axis_index("c")
        sid = lax.axis_index("s")
        wid = sid * 2 + core
        gb = wid * _EP

        pltpu.sync_copy(src_hbm.at[pl.ds(gb, _EP)], src_v.at[pl.ds(0, _EP)])
        st1 = pltpu.async_copy(dst_hbm.at[pl.ds(gb, _EP)],
                               dst_v.at[pl.ds(0, _EP)], stsem)
        st2 = pltpu.async_copy(h_hbm.at[pl.ds(gb, _EP)],
                               h_v.at[pl.ds(0, _EP)], stsem)
        if with_cnt:
            pltpu.sync_copy(ones_hbm, ones_v)

        @pl.when(sid < 15)
        def _():
            pltpu.sync_copy(zeros_hbm.at[pl.ds(sid * _ZB, _ZB)],
                            acc_s.at[pl.ds(sid * _ZB, _ZB)])
            if with_cnt:
                pltpu.sync_copy(zeros_hbm.at[pl.ds(sid * _ZB, _ZB)],
                                cnt_s.at[pl.ds(sid * _ZB, _ZB)])

        @pl.when(sid == 15)
        def _():
            pltpu.sync_copy(zeros_hbm.at[pl.ds(15 * _ZB, _N - 15 * _ZB)],
                            acc_s.at[pl.ds(15 * _ZB, _N - 15 * _ZB)])
            if with_cnt:
                pltpu.sync_copy(zeros_hbm.at[pl.ds(15 * _ZB, _N - 15 * _ZB)],
                                cnt_s.at[pl.ds(15 * _ZB, _N - 15 * _ZB)])

        iota16 = lax.iota(jnp.int32, 16)

        def splat(c):
            return jnp.full((16,), c, jnp.int32)

        def gview(eb, n, rows):
            return (p_hbm.at[src_v.at[pl.ds(eb, n)]],
                    rows.at[pl.ds(0, n)] if n != _CH else rows)

        def fetch(eb, rows, sem):
            s, d = gview(eb, _CH, rows)
            pltpu.async_copy(s, d, sem)

        def contract(eb, rows, ngroups, msg):
            for g in range(ngroups):
                r = g * 16 + iota16
                erow = eb + g * 16 + iota16
                accs = [plsc.load_gather(rows, [r, splat(8 * _H + o)])
                        for o in range(_H)]
                for k in range(_H):
                    hk = plsc.load_gather(h_v, [erow, splat(k)])
                    for o in range(_H):
                        accs[o] = accs[o] + hk * plsc.load_gather(
                            rows, [r, splat(8 * k + o)])
                for o in range(_H):
                    plsc.store_scatter(msg, [r, splat(o)], accs[o])

        def drain_scatter(msg, idxd, ssem):
            pltpu.make_async_copy(msg, acc_s.at[idxd], ssem).wait()
            if with_cnt:
                pltpu.make_async_copy(ones_v, cnt_s.at[idxd], ssem).wait()

        def compute_store(eb, rows, msg, idxd, gsem, ssem, drain):
            s, d = gview(eb, _CH, rows)
            pltpu.make_async_copy(s, d, gsem).wait()
            if drain is True:
                drain_scatter(msg, idxd, ssem)
            else:
                @pl.when(drain)
                def _():
                    drain_scatter(msg, idxd, ssem)
            for g in range(_CH // 16):
                idxd[pl.ds(g * 16, 16)] = dst_v[pl.ds(eb + g * 16, 16)]
            contract(eb, rows, _CH // 16, msg)
            pltpu.async_copy(msg, acc_s.at[idxd], ssem, add=True)
            if with_cnt:
                pltpu.async_copy(ones_v, cnt_s.at[idxd], ssem, add=True)

        fetch(0, rows_a, gsem_a)
        fetch(_CH, rows_b, gsem_b)
        st1.wait()
        st2.wait()
        plsc.subcore_barrier()

        def body(jj, carry):
            eb0 = 2 * jj * _CH
            compute_store(eb0, rows_a, msg_a, idxd_a, gsem_a, ssem_a, jj >= 1)
            fetch(eb0 + 2 * _CH, rows_a, gsem_a)
            compute_store(eb0 + _CH, rows_b, msg_b, idxd_b, gsem_b, ssem_b,
                          jj >= 1)

            @pl.when(jj < (_FC - 3) // 2)
            def _():
                fetch(eb0 + 3 * _CH, rows_b, gsem_b)

            return carry

        lax.fori_loop(0, (_FC - 1) // 2, body, 0)
        compute_store((_FC - 1) * _CH, rows_a, msg_a, idxd_a, gsem_a, ssem_a,
                      True)
        drain_scatter(msg_a, idxd_a, ssem_a)
        drain_scatter(msg_b, idxd_b, ssem_b)

        tb = _FC * _CH
        s, d = gview(tb, _TAIL, rows_a)
        pltpu.async_copy(s, d, gsem_a).wait()
        plsc.store_scatter(idxd8_v, [iota16], dst_v[pl.ds(tb, 16)],
                           mask=iota16 < _TAIL)
        contract(tb, rows_a, 1, msg_a)
        pltpu.sync_copy(msg_a.at[pl.ds(0, _TAIL)], acc_s.at[idxd8_v], add=True)
        if with_cnt:
            pltpu.sync_copy(ones_v.at[pl.ds(0, _TAIL)], cnt_s.at[idxd8_v], add=True)

        plsc.subcore_barrier()

        @pl.when(sid < 15)
        def _():
            pltpu.sync_copy(acc_s.at[pl.ds(sid * _ZB, _ZB)],
                            out_hbm.at[pl.ds(core * _N + sid * _ZB, _ZB)])
            if with_cnt:
                pltpu.sync_copy(cnt_s.at[pl.ds(sid * _ZB, _ZB)],
                                cnt_hbm.at[pl.ds(core * _N + sid * _ZB, _ZB)])

        @pl.when(sid == 15)
        def _():
            pltpu.sync_copy(acc_s.at[pl.ds(15 * _ZB, _N - 15 * _ZB)],
                            out_hbm.at[pl.ds(core * _N + 15 * _ZB, _N - 15 * _ZB)])
            if with_cnt:
                pltpu.sync_copy(cnt_s.at[pl.ds(15 * _ZB, _N - 15 * _ZB)],
                                cnt_hbm.at[pl.ds(core * _N + 15 * _ZB, _N - 15 * _ZB)])

    return layer_k


_sc_layer_cnt = _make_layer(True)
_sc_layer = _make_layer(False)


def _w2r(p, l, fin):
    return p['en2_W%d' % l].reshape(_H, fin, _H).transpose(1, 0, 2).reshape(fin, _H * _H)


def kernel(x, edge_index, edge_attr, batch, params):
    p = params
    src = edge_index[0]
    dst = edge_index[1]

    w1cat = jnp.concatenate([p['en1_W%d' % l] for l in range(_LAYERS)], axis=1)
    b1cat = jnp.concatenate([p['en1_b%d' % l] for l in range(_LAYERS)]).reshape(1, 3 * _H)

    fins = [_FIN, _H, _H]
    h0, h1, h2, P, R = _pre(edge_attr, w1cat, b1cat,
                            x, _w2r(p, 0, _FIN), p['en2_b0'].reshape(_FIN, _H),
                            p['root0'], p['bias0'].reshape(1, _H))
    hs = [h0, h1, h2]

    zerosN = jnp.zeros((_N, _H), _f32)
    onesC = jnp.ones((_CH, _H), _f32)

    cnt = None
    for l in range(_LAYERS):
        if l == 0:
            parts, cnt = _sc_layer_cnt(P, src, dst, hs[l], zerosN, onesC)
        else:
            parts = _sc_layer(P, src, dst, hs[l], zerosN)
        if l < _LAYERS - 1:
            fin = fins[l + 1]
            P, R = _combine_mid(parts, cnt, R,
                                _w2r(p, l + 1, fin),
                                p['en2_b%d' % (l + 1)].reshape(fin, _H),
                                p['root%d' % (l + 1)],
                                p['bias%d' % (l + 1)].reshape(1, _H))

    return _set2set(parts, cnt, R, batch.reshape(_N, 1),
                    p['Wih'], p['Whh'], p['lstm_b'].reshape(1, 4 * _H),
                    p['lin_W'], p['lin_b'].reshape(1, _T))
